# baseline (speedup 1.0000x reference)
import math

import jax
import jax.numpy as jnp
from jax import lax
from jax.experimental import pallas as pl
from jax.experimental.pallas import tpu as pltpu
from jax.experimental.pallas import tpu_sc as plsc

N_NODE = 50000
E = 320000
D = 128
H = 4
DH = D // H
INV_SQRT_DH = 1.0 / math.sqrt(DH)

NC = 2
NS = 16
L = 16
ESC = E // NC
SEG = 2000
NSEG = ESC // SEG
OWN = 4096
SUB = 512
NR = OWN // SUB
TRASH = SUB
PADV = 1 << 20
ACC_R = 516
W = 144
CAP = 160
CAPW = CAP * L
NOUT = 50176
ROW_BLK = 2000


def _proj_body(x_ref, wq_ref, bq_ref, wkv_ref, bkv_ref, q_ref, kv_ref):
    x = x_ref[...]
    q_ref[...] = jnp.dot(x, wq_ref[...], preferred_element_type=jnp.float32) + bq_ref[...]
    kv_ref[...] = jnp.dot(x, wkv_ref[...], preferred_element_type=jnp.float32) + bkv_ref[...]


def _proj(x, wq, bq, wkv, bkv):
    n = x.shape[0]
    return pl.pallas_call(
        _proj_body,
        grid=(n // ROW_BLK,),
        in_specs=[
            pl.BlockSpec((ROW_BLK, D), lambda i: (i, 0)),
            pl.BlockSpec((D, D), lambda i: (0, 0)),
            pl.BlockSpec((1, D), lambda i: (0, 0)),
            pl.BlockSpec((D, 2 * D), lambda i: (0, 0)),
            pl.BlockSpec((1, 2 * D), lambda i: (0, 0)),
        ],
        out_specs=[
            pl.BlockSpec((ROW_BLK, D), lambda i: (i, 0)),
            pl.BlockSpec((ROW_BLK, 2 * D), lambda i: (i, 0)),
        ],
        out_shape=[
            jax.ShapeDtypeStruct((n, D), jnp.float32),
            jax.ShapeDtypeStruct((n, 2 * D), jnp.float32),
        ],
    )(x, wq, bq, wkv, bkv)


def _edge_type_pass(src_hbm, dst_hbm, qtab, kvtab, out_hbm,
                    dstseg, srcseg, bq, bs, cnt, qidx, gki,
                    qrows, kvrows, acc, sem0, sem1):
    c = lax.axis_index("c")
    s = lax.axis_index("s")
    lane = lax.iota(jnp.int32, L)
    zero_i = jnp.zeros((L,), jnp.int32)
    ones_i = jnp.full((L,), 1, jnp.int32)
    zero_f = jnp.zeros((L,), jnp.float32)
    base_edge = c * ESC

    pad_vec = jnp.full((L,), PADV, jnp.int32)

    def fill(i, _):
        bq[pl.ds(i * L, L)] = pad_vec
        bs[pl.ds(i * L, L)] = zero_i
        return 0
    lax.fori_loop(0, NR * CAPW // L, fill, 0)
    for r in range(NR):
        cnt[pl.ds(r * L, L)] = zero_i

    def seg_body(t, _):
        pltpu.sync_copy(dst_hbm.at[pl.ds(base_edge + t * SEG, SEG)], dstseg)
        pltpu.sync_copy(src_hbm.at[pl.ds(base_edge + t * SEG, SEG)], srcseg)

        def bucketize(g, _):
            dvec = dstseg[pl.ds(g * L, L)]
            svec = srcseg[pl.ds(g * L, L)]
            m = lax.shift_right_logical(dvec, 12) == s
            rv = jnp.bitwise_and(lax.shift_right_logical(dvec, 9), NR - 1)
            cidx = rv * L + lane
            pos = plsc.load_gather(cnt, [cidx])
            plsc.addupdate_scatter(cnt, [cidx], ones_i, mask=m)
            pos = jnp.minimum(pos, CAP - 1)
            addr = rv * CAPW + pos * L + lane
            plsc.store_scatter(bq, [addr], dvec, mask=m)
            plsc.store_scatter(bs, [addr], svec, mask=m)
            return 0

        lax.fori_loop(0, SEG // L, bucketize, 0)
        return 0

    lax.fori_loop(0, NSEG, seg_body, 0)

    def round_body(r, _):
        def zrow(i, _):
            acc[pl.ds(i * L, L)] = zero_f
            return 0
        lax.fori_loop(0, ACC_R * W // L, zrow, 0)

        nmax = jnp.max(cnt[pl.ds(r * L, L)])

        def replay(j, _):
            vec = bq[pl.ds(r * CAPW + j * L, L)]
            locv = jnp.where(vec >= PADV, TRASH, jnp.bitwise_and(vec, SUB - 1))
            qidx[pl.ds(0, L)] = jnp.bitwise_and(vec, PADV - 1)
            gki[pl.ds(0, L)] = bs[pl.ds(r * CAPW + j * L, L)]
            cp_q = pltpu.async_copy(qtab.at[qidx], qrows, sem0)
            cp_kv = pltpu.async_copy(kvtab.at[gki], kvrows, sem1)
            cp_q.wait()
            cp_kv.wait()

            for e in range(L):
                svec = zero_f
                for h in range(H):
                    a0 = qrows[e, pl.ds(h * 32, L)] * kvrows[e, pl.ds(h * 32, L)]
                    a1 = qrows[e, pl.ds(h * 32 + L, L)] * kvrows[e, pl.ds(h * 32 + L, L)]
                    s_h = jnp.sum(a0 + a1)
                    svec = jnp.where(lane == h, s_h, svec)
                wvec = jnp.exp(svec)
                rbase = jnp.full((L,), locv[e] * W, jnp.int32) + lane
                for h in range(H):
                    w_h = wvec[h]
                    v0 = kvrows[e, pl.ds(D + h * 32, L)] * w_h
                    plsc.addupdate_scatter(acc, [rbase + h * 32], v0)
                    v1 = kvrows[e, pl.ds(D + h * 32 + L, L)] * w_h
                    plsc.addupdate_scatter(acc, [rbase + (h * 32 + L)], v1)
                plsc.addupdate_scatter(acc, [rbase + D], wvec, mask=lane < H)
            return 0

        lax.fori_loop(0, nmax, replay, 0)

        base_row = s * OWN + r * SUB

        @pl.when(base_row < N_NODE)
        def _():
            pltpu.sync_copy(acc.at[pl.ds(0, SUB * W)],
                            out_hbm.at[pl.ds(c * (NOUT * W) + base_row * W, SUB * W)])
        return 0

    lax.fori_loop(0, NR, round_body, 0)


def _sc_body(q_it, kv_us, src_ck, dst_ck, q_us, kv_it, src_cb, dst_cb,
             out_it, out_us,
             dstseg, srcseg, bq, bs, cnt, qidx, gki,
             qrows, kvrows, acc, sem0, sem1):
    scratches = (dstseg, srcseg, bq, bs, cnt, qidx, gki,
                 qrows, kvrows, acc, sem0, sem1)
    _edge_type_pass(src_ck, dst_ck, q_it, kv_us, out_it, *scratches)
    _edge_type_pass(src_cb, dst_cb, q_us, kv_it, out_us, *scratches)


def _sc_edge_agg(q_it, kv_us, src_ck, dst_ck, q_us, kv_it, src_cb, dst_cb):
    mesh = plsc.VectorSubcoreMesh(core_axis_name="c", subcore_axis_name="s",
                                  num_cores=NC, num_subcores=NS)
    f = pl.kernel(
        _sc_body,
        out_type=[jax.ShapeDtypeStruct((NC * NOUT * W,), jnp.float32)] * 2,
        mesh=mesh,
        compiler_params=pltpu.CompilerParams(needs_layout_passes=False),
        scratch_types=[
            pltpu.VMEM((SEG,), jnp.int32),
            pltpu.VMEM((SEG,), jnp.int32),
            pltpu.VMEM((NR * CAPW,), jnp.int32),
            pltpu.VMEM((NR * CAPW,), jnp.int32),
            pltpu.VMEM((NR * L,), jnp.int32),
            pltpu.VMEM((L,), jnp.int32),
            pltpu.VMEM((L,), jnp.int32),
            pltpu.VMEM((L, D), jnp.float32),
            pltpu.VMEM((L, 2 * D), jnp.float32),
            pltpu.VMEM((ACC_R * W,), jnp.float32),
            pltpu.SemaphoreType.DMA,
            pltpu.SemaphoreType.DMA,
        ],
    )
    return f(q_it, kv_us, src_ck, dst_ck, q_us, kv_it, src_cb, dst_cb)


def _out_body(x_ref, agg0_ref, agg1_ref, wout_ref, bout_ref, g_ref, b_ref, out_ref):
    blk = agg0_ref[...] + agg1_ref[...]
    vtmp = blk[:, :D]
    asum = blk[:, D:D + H]
    denom = jnp.clip(asum, 1e-9, None)
    vout = (vtmp.reshape(ROW_BLK, H, DH) / denom[:, :, None]).reshape(ROW_BLK, D)
    comb = jnp.dot(vout, wout_ref[...], preferred_element_type=jnp.float32) + bout_ref[...]
    y = x_ref[...] + comb
    mu = jnp.mean(y, axis=-1, keepdims=True)
    yc = y - mu
    var = jnp.mean(yc * yc, axis=-1, keepdims=True)
    out_ref[...] = yc * lax.rsqrt(var + 1e-5) * g_ref[...] + b_ref[...]


def _out_stage(x, agg0, agg1, wout, bout, g, b):
    n = x.shape[0]
    return pl.pallas_call(
        _out_body,
        grid=(n // ROW_BLK,),
        in_specs=[
            pl.BlockSpec((ROW_BLK, D), lambda i: (i, 0)),
            pl.BlockSpec((ROW_BLK, W), lambda i: (i, 0)),
            pl.BlockSpec((ROW_BLK, W), lambda i: (i, 0)),
            pl.BlockSpec((D, D), lambda i: (0, 0)),
            pl.BlockSpec((1, D), lambda i: (0, 0)),
            pl.BlockSpec((1, D), lambda i: (0, 0)),
            pl.BlockSpec((1, D), lambda i: (0, 0)),
        ],
        out_specs=pl.BlockSpec((ROW_BLK, D), lambda i: (i, 0)),
        out_shape=jax.ShapeDtypeStruct((n, D), jnp.float32),
    )(x, agg0, agg1, wout, bout, g, b)


def kernel(x_user, x_item, edge_index_clicks, edge_index_cb,
           Wq_clicks, Wk_clicks, Wv_clicks, bq_clicks, bk_clicks, bv_clicks, ea_clicks,
           Wq_cb, Wk_cb, Wv_cb, bq_cb, bk_cb, bv_cb, ea_cb,
           Wout_user, bout_user, etw_user, ln_g_user, ln_b_user,
           Wout_item, bout_item, etw_item, ln_g_item, ln_b_item):
    q_it, kv_it = _proj(x_item, Wq_clicks * INV_SQRT_DH,
                        (bq_clicks * INV_SQRT_DH)[None, :],
                        jnp.concatenate([Wk_cb, Wv_cb], axis=1),
                        jnp.concatenate([bk_cb, bv_cb])[None, :])
    q_us, kv_us = _proj(x_user, Wq_cb * INV_SQRT_DH,
                        (bq_cb * INV_SQRT_DH)[None, :],
                        jnp.concatenate([Wk_clicks, Wv_clicks], axis=1),
                        jnp.concatenate([bk_clicks, bv_clicks])[None, :])

    agg_it, agg_us = (a.reshape(NC, NOUT, W) for a in _sc_edge_agg(
        q_it, kv_us, edge_index_clicks[0], edge_index_clicks[1],
        q_us, kv_it, edge_index_cb[0], edge_index_cb[1]))
    it0, it1 = agg_it[0], agg_it[1]
    us0, us1 = agg_us[0], agg_us[1]

    out_item = _out_stage(x_item, it0, it1, Wout_item, bout_item[None, :],
                          ln_g_item[None, :], ln_b_item[None, :])
    out_user = _out_stage(x_user, us0, us1, Wout_user, bout_user[None, :],
                          ln_g_user[None, :], ln_b_user[None, :])
    return (out_user, out_item)


# speedup vs baseline: 1.0270x; 1.0270x over previous
import math

import jax
import jax.numpy as jnp
from jax import lax
from jax.experimental import pallas as pl
from jax.experimental.pallas import tpu as pltpu
from jax.experimental.pallas import tpu_sc as plsc

N_NODE = 50000
E = 320000
D = 128
H = 4
DH = D // H
INV_SQRT_DH = 1.0 / math.sqrt(DH)

NC = 2
NS = 16
L = 16
ESC = E // NC
SEG = 4000
NSEG = ESC // SEG
OWN = 4096
SUB = 512
NR = OWN // SUB
TRASH = SUB
SRCB = 17
ACC_R = 513
W = 144
CAP = 160
CAPW = (CAP + 2) * L
NOUT = 50176
ROW_BLK = 2000


def _proj_body(x_ref, wq_ref, bq_ref, wkv_ref, bkv_ref, q_ref, kv_ref):
    x = x_ref[...]
    q_ref[...] = jnp.dot(x, wq_ref[...], preferred_element_type=jnp.float32) + bq_ref[...]
    kv_ref[...] = jnp.dot(x, wkv_ref[...], preferred_element_type=jnp.float32) + bkv_ref[...]


def _proj(x, wq, bq, wkv, bkv):
    n = x.shape[0]
    return pl.pallas_call(
        _proj_body,
        grid=(n // ROW_BLK,),
        in_specs=[
            pl.BlockSpec((ROW_BLK, D), lambda i: (i, 0)),
            pl.BlockSpec((D, D), lambda i: (0, 0)),
            pl.BlockSpec((1, D), lambda i: (0, 0)),
            pl.BlockSpec((D, 2 * D), lambda i: (0, 0)),
            pl.BlockSpec((1, 2 * D), lambda i: (0, 0)),
        ],
        out_specs=[
            pl.BlockSpec((ROW_BLK, D), lambda i: (i, 0)),
            pl.BlockSpec((ROW_BLK, 2 * D), lambda i: (i, 0)),
        ],
        out_shape=[
            jax.ShapeDtypeStruct((n, D), jnp.float32),
            jax.ShapeDtypeStruct((n, 2 * D), jnp.float32),
        ],
    )(x, wq, bq, wkv, bkv)


def _edge_type_pass(src_hbm, dst_hbm, qtab, kvtab, out_hbm,
                    dstseg, srcseg, bq, cnt, qidx0, gki0, qidx1, gki1,
                    qrows0, kvrows0, qrows1, kvrows1, acc,
                    sem0, sem1, sem2, sem3):
    c = lax.axis_index("c")
    s = lax.axis_index("s")
    lane = lax.iota(jnp.int32, L)
    zero_i = jnp.zeros((L,), jnp.int32)
    ones_i = jnp.full((L,), 1, jnp.int32)
    zero_f = jnp.zeros((L,), jnp.float32)
    base_edge = c * ESC

    pad_vec = jnp.full((L,), TRASH << SRCB, jnp.int32)

    def fill(i, _):
        bq[pl.ds(i * L, L)] = pad_vec
        return 0
    lax.fori_loop(0, NR * CAPW // L, fill, 0)
    for r in range(NR):
        cnt[pl.ds(r * L, L)] = zero_i

    def seg_body(t, _):
        pltpu.sync_copy(dst_hbm.at[pl.ds(base_edge + t * SEG, SEG)], dstseg)
        pltpu.sync_copy(src_hbm.at[pl.ds(base_edge + t * SEG, SEG)], srcseg)

        def bucketize(g, _):
            dvec = dstseg[pl.ds(g * L, L)]
            svec = srcseg[pl.ds(g * L, L)]
            m = lax.shift_right_logical(dvec, 12) == s
            rv = jnp.bitwise_and(lax.shift_right_logical(dvec, 9), NR - 1)
            cidx = rv * L + lane
            pos = plsc.load_gather(cnt, [cidx])
            plsc.addupdate_scatter(cnt, [cidx], ones_i, mask=m)
            pos = jnp.minimum(pos, CAP - 1)
            addr = rv * CAPW + pos * L + lane
            loc = jnp.bitwise_and(dvec, SUB - 1)
            packed = jnp.bitwise_or(lax.shift_left(loc, SRCB), svec)
            plsc.store_scatter(bq, [addr], packed, mask=m)
            return 0

        lax.fori_loop(0, SEG // L, bucketize, 0)
        return 0

    lax.fori_loop(0, NSEG, seg_body, 0)

    def round_body(r, _):
        def zrow(i, _):
            acc[pl.ds(i * L, L)] = zero_f
            return 0
        lax.fori_loop(0, ACC_R * W // L, zrow, 0)

        nmax = jnp.max(cnt[pl.ds(r * L, L)])

        bufs = ((qidx0, gki0, qrows0, kvrows0, sem0, sem1),
                (qidx1, gki1, qrows1, kvrows1, sem2, sem3))

        def issue(j, buf):
            qx, gx, qb, kb, sq, sk = buf
            vec = bq[pl.ds(r * CAPW + j * L, L)]
            locv = lax.shift_right_logical(vec, SRCB)
            qa = jnp.minimum(s * OWN + r * SUB + locv, N_NODE - 1)
            qx[pl.ds(0, L)] = qa
            gx[pl.ds(0, L)] = jnp.bitwise_and(vec, (1 << SRCB) - 1)
            pltpu.async_copy(qtab.at[qx], qb, sq)
            pltpu.async_copy(kvtab.at[gx], kb, sk)

        def drain(buf):
            qx, gx, qb, kb, sq, sk = buf
            pltpu.make_async_copy(qtab.at[qx], qb, sq).wait()
            pltpu.make_async_copy(kvtab.at[gx], kb, sk).wait()

        def compute(j, buf):
            qx, gx, qb, kb, sq, sk = buf
            drain(buf)
            vec = bq[pl.ds(r * CAPW + j * L, L)]
            locv = lax.shift_right_logical(vec, SRCB)
            for e in range(L):
                svec = zero_f
                for h in range(H):
                    a0 = qb[e, pl.ds(h * 32, L)] * kb[e, pl.ds(h * 32, L)]
                    a1 = qb[e, pl.ds(h * 32 + L, L)] * kb[e, pl.ds(h * 32 + L, L)]
                    s_h = jnp.sum(a0 + a1)
                    svec = jnp.where(lane == h, s_h, svec)
                wvec = jnp.exp(svec)
                rbase = jnp.full((L,), locv[e] * W, jnp.int32) + lane
                for h in range(H):
                    w_h = wvec[h]
                    v0 = kb[e, pl.ds(D + h * 32, L)] * w_h
                    plsc.addupdate_scatter(acc, [rbase + h * 32], v0)
                    v1 = kb[e, pl.ds(D + h * 32 + L, L)] * w_h
                    plsc.addupdate_scatter(acc, [rbase + (h * 32 + L)], v1)
                plsc.addupdate_scatter(acc, [rbase + D], wvec, mask=lane < H)

        issue(0, bufs[0])
        issue(1, bufs[1])

        def replay2(j2, _):
            compute(2 * j2, bufs[0])
            issue(2 * j2 + 2, bufs[0])
            compute(2 * j2 + 1, bufs[1])
            issue(2 * j2 + 3, bufs[1])
            return 0

        lax.fori_loop(0, (nmax + 1) // 2, replay2, 0)
        drain(bufs[0])
        drain(bufs[1])

        base_row = s * OWN + r * SUB

        @pl.when(base_row < N_NODE)
        def _():
            pltpu.sync_copy(acc.at[pl.ds(0, SUB * W)],
                            out_hbm.at[pl.ds(c * (NOUT * W) + base_row * W, SUB * W)])
        return 0

    lax.fori_loop(0, NR, round_body, 0)


def _sc_body(q_it, kv_us, src_ck, dst_ck, q_us, kv_it, src_cb, dst_cb,
             out_it, out_us,
             dstseg, srcseg, bq, cnt, qidx0, gki0, qidx1, gki1,
             qrows0, kvrows0, qrows1, kvrows1, acc,
             sem0, sem1, sem2, sem3):
    scratches = (dstseg, srcseg, bq, cnt, qidx0, gki0, qidx1, gki1,
                 qrows0, kvrows0, qrows1, kvrows1, acc,
                 sem0, sem1, sem2, sem3)
    _edge_type_pass(src_ck, dst_ck, q_it, kv_us, out_it, *scratches)
    _edge_type_pass(src_cb, dst_cb, q_us, kv_it, out_us, *scratches)


def _sc_edge_agg(q_it, kv_us, src_ck, dst_ck, q_us, kv_it, src_cb, dst_cb):
    mesh = plsc.VectorSubcoreMesh(core_axis_name="c", subcore_axis_name="s",
                                  num_cores=NC, num_subcores=NS)
    f = pl.kernel(
        _sc_body,
        out_type=[jax.ShapeDtypeStruct((NC * NOUT * W,), jnp.float32)] * 2,
        mesh=mesh,
        compiler_params=pltpu.CompilerParams(needs_layout_passes=False),
        scratch_types=[
            pltpu.VMEM((SEG,), jnp.int32),
            pltpu.VMEM((SEG,), jnp.int32),
            pltpu.VMEM((NR * CAPW,), jnp.int32),
            pltpu.VMEM((NR * L,), jnp.int32),
            pltpu.VMEM((L,), jnp.int32),
            pltpu.VMEM((L,), jnp.int32),
            pltpu.VMEM((L,), jnp.int32),
            pltpu.VMEM((L,), jnp.int32),
            pltpu.VMEM((L, D), jnp.float32),
            pltpu.VMEM((L, 2 * D), jnp.float32),
            pltpu.VMEM((L, D), jnp.float32),
            pltpu.VMEM((L, 2 * D), jnp.float32),
            pltpu.VMEM((ACC_R * W,), jnp.float32),
            pltpu.SemaphoreType.DMA,
            pltpu.SemaphoreType.DMA,
            pltpu.SemaphoreType.DMA,
            pltpu.SemaphoreType.DMA,
        ],
    )
    return f(q_it, kv_us, src_ck, dst_ck, q_us, kv_it, src_cb, dst_cb)


def _out_body(x_ref, agg0_ref, agg1_ref, wout_ref, bout_ref, g_ref, b_ref, out_ref):
    blk = agg0_ref[...] + agg1_ref[...]
    vtmp = blk[:, :D]
    asum = blk[:, D:D + H]
    denom = jnp.clip(asum, 1e-9, None)
    vout = (vtmp.reshape(ROW_BLK, H, DH) / denom[:, :, None]).reshape(ROW_BLK, D)
    comb = jnp.dot(vout, wout_ref[...], preferred_element_type=jnp.float32) + bout_ref[...]
    y = x_ref[...] + comb
    mu = jnp.mean(y, axis=-1, keepdims=True)
    yc = y - mu
    var = jnp.mean(yc * yc, axis=-1, keepdims=True)
    out_ref[...] = yc * lax.rsqrt(var + 1e-5) * g_ref[...] + b_ref[...]


def _out_stage(x, agg0, agg1, wout, bout, g, b):
    n = x.shape[0]
    return pl.pallas_call(
        _out_body,
        grid=(n // ROW_BLK,),
        in_specs=[
            pl.BlockSpec((ROW_BLK, D), lambda i: (i, 0)),
            pl.BlockSpec((ROW_BLK, W), lambda i: (i, 0)),
            pl.BlockSpec((ROW_BLK, W), lambda i: (i, 0)),
            pl.BlockSpec((D, D), lambda i: (0, 0)),
            pl.BlockSpec((1, D), lambda i: (0, 0)),
            pl.BlockSpec((1, D), lambda i: (0, 0)),
            pl.BlockSpec((1, D), lambda i: (0, 0)),
        ],
        out_specs=pl.BlockSpec((ROW_BLK, D), lambda i: (i, 0)),
        out_shape=jax.ShapeDtypeStruct((n, D), jnp.float32),
    )(x, agg0, agg1, wout, bout, g, b)


def kernel(x_user, x_item, edge_index_clicks, edge_index_cb,
           Wq_clicks, Wk_clicks, Wv_clicks, bq_clicks, bk_clicks, bv_clicks, ea_clicks,
           Wq_cb, Wk_cb, Wv_cb, bq_cb, bk_cb, bv_cb, ea_cb,
           Wout_user, bout_user, etw_user, ln_g_user, ln_b_user,
           Wout_item, bout_item, etw_item, ln_g_item, ln_b_item):
    q_it, kv_it = _proj(x_item, Wq_clicks * INV_SQRT_DH,
                        (bq_clicks * INV_SQRT_DH)[None, :],
                        jnp.concatenate([Wk_cb, Wv_cb], axis=1),
                        jnp.concatenate([bk_cb, bv_cb])[None, :])
    q_us, kv_us = _proj(x_user, Wq_cb * INV_SQRT_DH,
                        (bq_cb * INV_SQRT_DH)[None, :],
                        jnp.concatenate([Wk_clicks, Wv_clicks], axis=1),
                        jnp.concatenate([bk_clicks, bv_clicks])[None, :])

    agg_it, agg_us = (a.reshape(NC, NOUT, W) for a in _sc_edge_agg(
        q_it, kv_us, edge_index_clicks[0], edge_index_clicks[1],
        q_us, kv_it, edge_index_cb[0], edge_index_cb[1]))
    it0, it1 = agg_it[0], agg_it[1]
    us0, us1 = agg_us[0], agg_us[1]

    out_item = _out_stage(x_item, it0, it1, Wout_item, bout_item[None, :],
                          ln_g_item[None, :], ln_b_item[None, :])
    out_user = _out_stage(x_user, us0, us1, Wout_user, bout_user[None, :],
                          ln_g_user[None, :], ln_b_user[None, :])
    return (out_user, out_item)
